# Initial kernel scaffold; baseline (speedup 1.0000x reference)
import jax
import jax.numpy as jnp
from jax.experimental import pallas as pl


def kernel(x, A_fwd, A_bwd, E1, E2, Wg1, Wg2, Wzr, bzr, Wc, bc):
    raise NotImplementedError("write your pallas kernel here")


# speedup vs baseline: 3.1465x; 3.1465x over previous
import functools

import jax
import jax.numpy as jnp
from jax.experimental import pallas as pl
from jax.experimental.pallas import tpu as pltpu

_ALPHA = 0.05
_BETA = 3.0
_K = 2
_EPS = 1e-8


def _encoder_body(x_ref, af_ref, ab_ref, e1_ref, e2_ref, wg1_ref, wg2_ref,
                  wzr_ref, bzr_ref, wc_ref, bc_ref, out_ref, *, P, N, H):
    f32 = jnp.float32

    Af = jnp.maximum(af_ref[...], 0.0)
    Afn = Af / (jnp.sum(Af, axis=-1, keepdims=True) + _EPS)
    Ab = jnp.maximum(ab_ref[...], 0.0)
    Abn = Ab / (jnp.sum(Ab, axis=-1, keepdims=True) + _EPS)

    e1 = e1_ref[...]
    e2 = e2_ref[...]
    wg1 = wg1_ref[...]
    wg2 = wg2_ref[...]
    wzr = wzr_ref[...]
    bzr_v = bzr_ref[...]
    wc = wc_ref[...]
    bc_v = bc_ref[...]

    def gconv(y, sup, W, bvec):
        outs = [y]
        for A in sup:
            hh = y
            for _ in range(_K):
                hh = _ALPHA * y + (1.0 - _ALPHA) * jnp.dot(
                    A, hh, preferred_element_type=f32)
                outs.append(hh)
        ho = jnp.concatenate(outs, axis=-1)
        return jnp.dot(ho, W, preferred_element_type=f32) + bvec

    def step(t, h):
        xt = x_ref[0, t]
        inp = jnp.concatenate([xt, h], axis=-1)
        f1 = jnp.tanh(jnp.dot(inp, wg1, preferred_element_type=f32) * e1)
        f2 = jnp.tanh(jnp.dot(inp, wg2, preferred_element_type=f32) * e2)
        M1 = jax.lax.dot_general(f1, f2, (((1,), (1,)), ((), ())),
                                 preferred_element_type=f32)
        M2 = jax.lax.dot_general(f2, f1, (((1,), (1,)), ((), ())),
                                 preferred_element_type=f32)
        Tm = jnp.tanh(_BETA * (M1 - M2))
        Ar = jnp.maximum(Tm, 0.0)
        Ac = jnp.maximum(-Tm, 0.0)
        Ad1 = Ar / (jnp.sum(Ar, axis=-1, keepdims=True) + _EPS)
        Ad2 = Ac / (jnp.sum(Ac, axis=-1, keepdims=True) + _EPS)
        sup = (Afn, Abn, Ad1, Ad2)
        zr = jax.nn.sigmoid(gconv(inp, sup, wzr, bzr_v))
        z = zr[:, :H]
        r = zr[:, H:]
        cin = jnp.concatenate([xt, r * h], axis=-1)
        c = jnp.tanh(gconv(cin, sup, wc, bc_v))
        return z * h + (1.0 - z) * c

    h0 = jnp.zeros((N, H), dtype=f32)
    hf = jax.lax.fori_loop(0, P, step, h0)
    out_ref[0] = hf


def kernel(x, A_fwd, A_bwd, E1, E2, Wg1, Wg2, Wzr, bzr, Wc, bc):
    B, P, N, C = x.shape
    H = Wc.shape[1]
    EMB = E1.shape[1]
    D = C + H
    feat = Wzr.shape[0]

    bzr2 = bzr.reshape(1, -1)
    bc2 = bc.reshape(1, -1)

    body = functools.partial(_encoder_body, P=P, N=N, H=H)
    grid = (B,)
    out = pl.pallas_call(
        body,
        grid=grid,
        in_specs=[
            pl.BlockSpec((1, P, N, C), lambda b: (b, 0, 0, 0)),
            pl.BlockSpec((N, N), lambda b: (0, 0)),
            pl.BlockSpec((N, N), lambda b: (0, 0)),
            pl.BlockSpec((N, EMB), lambda b: (0, 0)),
            pl.BlockSpec((N, EMB), lambda b: (0, 0)),
            pl.BlockSpec((D, EMB), lambda b: (0, 0)),
            pl.BlockSpec((D, EMB), lambda b: (0, 0)),
            pl.BlockSpec((feat, 2 * H), lambda b: (0, 0)),
            pl.BlockSpec((1, 2 * H), lambda b: (0, 0)),
            pl.BlockSpec((feat, H), lambda b: (0, 0)),
            pl.BlockSpec((1, H), lambda b: (0, 0)),
        ],
        out_specs=pl.BlockSpec((1, N, H), lambda b: (b, 0, 0)),
        out_shape=jax.ShapeDtypeStruct((B, N, H), x.dtype),
        compiler_params=pltpu.CompilerParams(
            dimension_semantics=("parallel",),
        ),
    )(x, A_fwd, A_bwd, E1, E2, Wg1, Wg2, Wzr, bzr2, Wc, bc2)
    return out
